# baseline (speedup 1.0000x reference)
import functools

import jax
import jax.numpy as jnp
from jax.experimental import pallas as pl
from jax.experimental.pallas import tpu as pltpu


def _sims_body(ft0h_ref, ft0l_ref, ft1_ref, out_ref, *, scale):
    b = ft1_ref[...]
    b_hi = b.astype(jnp.bfloat16)
    b_lo = (b - b_hi.astype(jnp.float32)).astype(jnp.bfloat16)
    dn = (((1,), (1,)), ((), ()))
    raw = jax.lax.dot_general(ft0h_ref[...], b_hi,
                              dimension_numbers=dn,
                              preferred_element_type=jnp.float32)
    raw += jax.lax.dot_general(ft0h_ref[...], b_lo,
                               dimension_numbers=dn,
                               preferred_element_type=jnp.float32)
    raw += jax.lax.dot_general(ft0l_ref[...], b_hi,
                               dimension_numbers=dn,
                               preferred_element_type=jnp.float32)
    out_ref[...] = raw * scale + 0.5


def _post_body(sims_ref, pt0f_ref, pt0t_ref, pt1t_ref,
               simmax_ref, ratio_ref, cyc_ref, ixs_ref):
    sims = sims_ref[...]
    q, k = sims.shape

    rowmax = jnp.max(sims, axis=1, keepdims=True)
    colid = jax.lax.broadcasted_iota(jnp.int32, (q, k), 1)
    ixs = jnp.min(jnp.where(sims == rowmax, colid, k), axis=1,
                  keepdims=True)

    onehot = colid == ixs
    pt1x = pt1t_ref[0:1, :]
    pt1y = pt1t_ref[1:2, :]
    pt1mx = jnp.max(jnp.where(onehot, pt1x, -1.0), axis=1, keepdims=True)
    pt1my = jnp.max(jnp.where(onehot, pt1y, -1.0), axis=1, keepdims=True)

    near = (jnp.abs(pt1x - pt1mx) < 64.0) & (jnp.abs(pt1y - pt1my) < 64.0)
    sim_reverse = jnp.max(jnp.where(near, 0.0, sims), axis=1, keepdims=True)

    colmax = jnp.max(sims, axis=0, keepdims=True)
    rowid = jax.lax.broadcasted_iota(jnp.int32, (q, k), 0)
    colarg = jnp.min(jnp.where(sims == colmax, rowid, q), axis=0,
                     keepdims=True)
    ix_cyc = jnp.min(jnp.where(onehot, colarg, q), axis=1,
                     keepdims=True)

    qid = jax.lax.broadcasted_iota(jnp.int32, (q, q), 1)
    onehot2 = qid == ix_cyc
    pt0x = pt0t_ref[0:1, :]
    pt0y = pt0t_ref[1:2, :]
    pt0cx = jnp.max(jnp.where(onehot2, pt0x, -1.0), axis=1, keepdims=True)
    pt0cy = jnp.max(jnp.where(onehot2, pt0y, -1.0), axis=1, keepdims=True)
    dx = pt0cx - pt0f_ref[:, 0:1]
    dy = pt0cy - pt0f_ref[:, 1:2]

    simmax_ref[...] = rowmax
    ratio_ref[...] = rowmax / sim_reverse
    cyc_ref[...] = jnp.sqrt(dx * dx + dy * dy)
    ixs_ref[...] = ixs


def kernel(ft0, ft1, pt0, pt1):
    nq, c, h, w = ft0.shape
    nk = ft1.shape[0]
    d = c * h * w
    scale = 1.0 / (h ** 2) / 2.0

    ft0f = ft0.reshape(nq, d)
    ft1f = ft1.reshape(nk, d)
    ft0h = ft0f.astype(jnp.bfloat16)
    ft0l = (ft0f - ft0h.astype(jnp.float32)).astype(jnp.bfloat16)

    nblk = 256 if nk % 256 == 0 else nk
    sims = pl.pallas_call(
        functools.partial(_sims_body, scale=scale),
        grid=(nk // nblk,),
        in_specs=[
            pl.BlockSpec((nq, d), lambda i: (0, 0)),
            pl.BlockSpec((nq, d), lambda i: (0, 0)),
            pl.BlockSpec((nblk, d), lambda i: (i, 0)),
        ],
        out_specs=pl.BlockSpec((nq, nblk), lambda i: (0, i)),
        out_shape=jax.ShapeDtypeStruct((nq, nk), jnp.float32),
    )(ft0h, ft0l, ft1f)

    pt0f = pt0.astype(jnp.float32)
    pt0t = pt0f.T
    pt1t = pt1.astype(jnp.float32).T

    simmax, ratios, cyc, ixs = pl.pallas_call(
        _post_body,
        out_shape=(
            jax.ShapeDtypeStruct((nq, 1), jnp.float32),
            jax.ShapeDtypeStruct((nq, 1), jnp.float32),
            jax.ShapeDtypeStruct((nq, 1), jnp.float32),
            jax.ShapeDtypeStruct((nq, 1), jnp.int32),
        ),
    )(sims, pt0f, pt0t, pt1t)

    return (simmax.reshape(nq), ratios.reshape(nq),
            cyc.reshape(nq), ixs.reshape(nq))


# speedup vs baseline: 3.2737x; 3.2737x over previous
import functools

import jax
import jax.numpy as jnp
from jax.experimental import pallas as pl
from jax.experimental.pallas import tpu as pltpu


def _fused_body(ft0t_ref, ft1t_ref, pt0f_ref, pt0t_ref, pt1t_ref,
                simmax_ref, ratio_ref, cyc_ref, ixs_ref,
                sims_ref, rowmax_ref, rowarg_ref, colarg_ref,
                *, scale, nblk, nb):
    i = pl.program_id(0)

    @pl.when(i < nb)
    def _compute():
        raw = jax.lax.dot_general(
            ft0t_ref[...], ft1t_ref[...],
            dimension_numbers=(((0,), (0,)), ((), ())),
            preferred_element_type=jnp.float32,
        )
        s = raw * scale + 0.5
        q = s.shape[0]
        sims_ref[:, pl.ds(i * nblk, nblk)] = s

        bcolid = jax.lax.broadcasted_iota(jnp.int32, (q, nblk), 1)
        bmax = jnp.max(s, axis=1, keepdims=True)
        barg = jnp.min(jnp.where(s == bmax, bcolid, nblk), axis=1,
                       keepdims=True) + i * nblk
        prevmax = jnp.where(i == 0, -jnp.inf, rowmax_ref[...])
        better = bmax > prevmax
        rowmax_ref[...] = jnp.where(better, bmax, prevmax)
        rowarg_ref[...] = jnp.where(better, barg, rowarg_ref[...])

        browid = jax.lax.broadcasted_iota(jnp.int32, (q, nblk), 0)
        cmax = jnp.max(s, axis=0, keepdims=True)
        carg = jnp.min(jnp.where(s == cmax, browid, q), axis=0,
                       keepdims=True)
        colarg_ref[0:1, pl.ds(i * nblk, nblk)] = carg

    @pl.when(i == nb)
    def _finalize():
        sims = sims_ref[...]
        q, k = sims.shape
        ixs = rowarg_ref[...]
        rowmax = rowmax_ref[...]

        colid = jax.lax.broadcasted_iota(jnp.int32, (q, k), 1)
        onehot = colid == ixs
        pt1x = pt1t_ref[0:1, :]
        pt1y = pt1t_ref[1:2, :]
        pt1mx = jnp.max(jnp.where(onehot, pt1x, -1.0), axis=1, keepdims=True)
        pt1my = jnp.max(jnp.where(onehot, pt1y, -1.0), axis=1, keepdims=True)

        near = (jnp.abs(pt1x - pt1mx) < 64.0) & (jnp.abs(pt1y - pt1my) < 64.0)
        sim_rev = jnp.max(jnp.where(near, 0.0, sims), axis=1, keepdims=True)

        ix_cyc = jnp.min(jnp.where(onehot, colarg_ref[...], q), axis=1,
                         keepdims=True)

        qid = jax.lax.broadcasted_iota(jnp.int32, (q, q), 1)
        onehot2 = qid == ix_cyc
        pt0x = pt0t_ref[0:1, :]
        pt0y = pt0t_ref[1:2, :]
        pt0cx = jnp.max(jnp.where(onehot2, pt0x, -1.0), axis=1, keepdims=True)
        pt0cy = jnp.max(jnp.where(onehot2, pt0y, -1.0), axis=1, keepdims=True)
        dx = pt0cx - pt0f_ref[:, 0:1]
        dy = pt0cy - pt0f_ref[:, 1:2]

        simmax_ref[...] = rowmax
        ratio_ref[...] = rowmax / sim_rev
        cyc_ref[...] = jnp.sqrt(dx * dx + dy * dy)
        ixs_ref[...] = ixs


def kernel(ft0, ft1, pt0, pt1):
    nq, c, h, w = ft0.shape
    nk = ft1.shape[0]
    d = c * h * w
    scale = 1.0 / (h ** 2) / 2.0

    ft0t = jax.lax.transpose(ft0, (1, 2, 3, 0)).reshape(d, nq)
    ft1t = jax.lax.transpose(ft1, (1, 2, 3, 0)).reshape(d, nk)

    pt0f = pt0.astype(jnp.float32)
    pt0t = pt0f.T
    pt1t = pt1.astype(jnp.float32).T

    nblk = 256 if nk % 256 == 0 else nk
    nb = nk // nblk
    last = nb - 1

    out_shapes = (
        jax.ShapeDtypeStruct((nq, 1), jnp.float32),
        jax.ShapeDtypeStruct((nq, 1), jnp.float32),
        jax.ShapeDtypeStruct((nq, 1), jnp.float32),
        jax.ShapeDtypeStruct((nq, 1), jnp.int32),
    )
    small = pl.BlockSpec((nq, 1), lambda i: (0, 0))
    simmax, ratios, cyc, ixs = pl.pallas_call(
        functools.partial(_fused_body, scale=scale, nblk=nblk, nb=nb),
        grid=(nb + 1,),
        in_specs=[
            pl.BlockSpec((d, nq), lambda i: (0, 0)),
            pl.BlockSpec((d, nblk), lambda i: (0, jnp.minimum(i, last))),
            pl.BlockSpec((nq, 2), lambda i: (0, 0)),
            pl.BlockSpec((2, nq), lambda i: (0, 0)),
            pl.BlockSpec((2, nk), lambda i: (0, 0)),
        ],
        out_specs=(small, small, small, small),
        out_shape=out_shapes,
        scratch_shapes=[
            pltpu.VMEM((nq, nk), jnp.float32),
            pltpu.VMEM((nq, 1), jnp.float32),
            pltpu.VMEM((nq, 1), jnp.int32),
            pltpu.VMEM((1, nk), jnp.int32),
        ],
    )(ft0t, ft1t, pt0f, pt0t, pt1t)

    return (simmax.reshape(nq), ratios.reshape(nq),
            cyc.reshape(nq), ixs.reshape(nq))


# speedup vs baseline: 3.9478x; 1.2059x over previous
import functools

import jax
import jax.numpy as jnp
from jax.experimental import pallas as pl
from jax.experimental.pallas import tpu as pltpu


def _fused_body(ft0t_ref, ft1t_ref, pt0t_ref, pt1t_ref,
                out_ref,
                sims_ref, rowstat_ref, colarg_ref,
                *, scale, nblk, nb, dh):
    i = pl.program_id(0)
    kk = jax.lax.rem(i, 2)
    j = jax.lax.div(i, 2)

    @pl.when(i < 2 * nb)
    def _compute():
        part = jax.lax.dot_general(
            ft0t_ref[pl.ds(kk * dh, dh), :], ft1t_ref[...],
            dimension_numbers=(((0,), (0,)), ((), ())),
            preferred_element_type=jnp.float32,
        )
        q = part.shape[0]

        @pl.when(kk == 0)
        def _store_partial():
            sims_ref[:, pl.ds(j * nblk, nblk)] = part

        @pl.when(kk == 1)
        def _finish_block():
            s = (sims_ref[:, pl.ds(j * nblk, nblk)] + part) * scale + 0.5
            sims_ref[:, pl.ds(j * nblk, nblk)] = s

            bcolid = jax.lax.broadcasted_iota(jnp.int32, (q, nblk), 1)
            bmax = jnp.max(s, axis=1, keepdims=True)
            barg = (jnp.min(jnp.where(s == bmax, bcolid, nblk), axis=1,
                            keepdims=True) + j * nblk).astype(jnp.float32)
            prevmax = jnp.where(j == 0, -jnp.inf, rowstat_ref[:, 0:1])
            better = bmax > prevmax
            rowstat_ref[:, 0:1] = jnp.where(better, bmax, prevmax)
            rowstat_ref[:, 1:2] = jnp.where(better, barg, rowstat_ref[:, 1:2])

            browid = jax.lax.broadcasted_iota(jnp.int32, (q, nblk), 0)
            cmax = jnp.max(s, axis=0, keepdims=True)
            carg = jnp.min(jnp.where(s == cmax, browid, q), axis=0,
                           keepdims=True)
            colarg_ref[0:1, pl.ds(j * nblk, nblk)] = carg

    @pl.when(i == 2 * nb)
    def _finalize():
        sims = sims_ref[...]
        q, k = sims.shape
        ixs = rowstat_ref[:, 1:2].astype(jnp.int32)
        rowmax = rowstat_ref[:, 0:1]

        colid = jax.lax.broadcasted_iota(jnp.int32, (q, k), 1)
        onehot = colid == ixs
        pt1x = pt1t_ref[0:1, :]
        pt1y = pt1t_ref[1:2, :]
        pt1mx = jnp.max(jnp.where(onehot, pt1x, -1.0), axis=1, keepdims=True)
        pt1my = jnp.max(jnp.where(onehot, pt1y, -1.0), axis=1, keepdims=True)

        near = (jnp.abs(pt1x - pt1mx) < 64.0) & (jnp.abs(pt1y - pt1my) < 64.0)
        sim_rev = jnp.max(jnp.where(near, 0.0, sims), axis=1, keepdims=True)

        ix_cyc = jnp.min(jnp.where(onehot, colarg_ref[...], q), axis=1,
                         keepdims=True)

        qid = jax.lax.broadcasted_iota(jnp.int32, (q, q), 1)
        onehot2 = qid == ix_cyc
        pt0x = pt0t_ref[0:1, :]
        pt0y = pt0t_ref[1:2, :]
        pt0cx = jnp.max(jnp.where(onehot2, pt0x, -1.0), axis=1, keepdims=True)
        pt0cy = jnp.max(jnp.where(onehot2, pt0y, -1.0), axis=1, keepdims=True)
        dx = pt0cx - jax.lax.transpose(pt0x, (1, 0))
        dy = pt0cy - jax.lax.transpose(pt0y, (1, 0))

        out_ref[:, 0:1] = rowmax
        out_ref[:, 1:2] = rowmax / sim_rev
        out_ref[:, 2:3] = jnp.sqrt(dx * dx + dy * dy)
        out_ref[:, 3:4] = jax.lax.bitcast_convert_type(ixs, jnp.float32)


def kernel(ft0, ft1, pt0, pt1):
    nq, c, h, w = ft0.shape
    nk = ft1.shape[0]
    d = c * h * w
    scale = 1.0 / (h ** 2) / 2.0

    ft0t = jax.lax.transpose(ft0, (1, 2, 3, 0)).reshape(d, nq)
    ft1t = jax.lax.transpose(ft1, (1, 2, 3, 0)).reshape(d, nk)

    pt0t = pt0.astype(jnp.float32).T
    pt1t = pt1.astype(jnp.float32).T

    nblk = 512 if nk % 512 == 0 else nk
    nb = nk // nblk
    dh = d // 2
    last = nb - 1

    out = pl.pallas_call(
        functools.partial(_fused_body, scale=scale, nblk=nblk, nb=nb, dh=dh),
        grid=(2 * nb + 1,),
        in_specs=[
            pl.BlockSpec((d, nq), lambda i: (0, 0)),
            pl.BlockSpec(
                (dh, nblk),
                lambda i: (jax.lax.rem(i, 2),
                           jnp.minimum(jax.lax.div(i, 2), last))),
            pl.BlockSpec((2, nq), lambda i: (0, 0)),
            pl.BlockSpec((2, nk), lambda i: (0, 0)),
        ],
        out_specs=pl.BlockSpec((nq, 4), lambda i: (0, 0)),
        out_shape=jax.ShapeDtypeStruct((nq, 4), jnp.float32),
        scratch_shapes=[
            pltpu.VMEM((nq, nk), jnp.float32),
            pltpu.VMEM((nq, 2), jnp.float32),
            pltpu.VMEM((1, nk), jnp.int32),
        ],
    )(ft0t, ft1t, pt0t, pt1t)

    return (out[:, 0], out[:, 1], out[:, 2],
            jax.lax.bitcast_convert_type(out[:, 3], jnp.int32))


# speedup vs baseline: 3.9743x; 1.0067x over previous
import functools

import jax
import jax.numpy as jnp
from jax.experimental import pallas as pl
from jax.experimental.pallas import tpu as pltpu


def _fused_body(ft0t_ref, ft1t_ref, pt0t_ref, pt1t_ref,
                out_ref,
                sims_ref, rowstat_ref, colarg_ref,
                *, scale, nblk, nb, dh, ks):
    i = pl.program_id(0)
    kk = jax.lax.rem(i, ks)
    j = jax.lax.div(i, ks)

    @pl.when(i < ks * nb)
    def _compute():
        part = jax.lax.dot_general(
            ft0t_ref[pl.ds(kk * dh, dh), :], ft1t_ref[...],
            dimension_numbers=(((0,), (0,)), ((), ())),
            preferred_element_type=jnp.float32,
        )
        q = part.shape[0]

        @pl.when(kk == 0)
        def _store_partial():
            sims_ref[:, pl.ds(j * nblk, nblk)] = part

        @pl.when((kk > 0) & (kk < ks - 1))
        def _accum_partial():
            sims_ref[:, pl.ds(j * nblk, nblk)] += part

        @pl.when(kk == ks - 1)
        def _finish_block():
            s = (sims_ref[:, pl.ds(j * nblk, nblk)] + part) * scale + 0.5
            sims_ref[:, pl.ds(j * nblk, nblk)] = s

            bcolid = jax.lax.broadcasted_iota(jnp.int32, (q, nblk), 1)
            bmax = jnp.max(s, axis=1, keepdims=True)
            barg = (jnp.min(jnp.where(s == bmax, bcolid, nblk), axis=1,
                            keepdims=True) + j * nblk).astype(jnp.float32)
            prevmax = jnp.where(j == 0, -jnp.inf, rowstat_ref[:, 0:1])
            better = bmax > prevmax
            rowstat_ref[:, 0:1] = jnp.where(better, bmax, prevmax)
            rowstat_ref[:, 1:2] = jnp.where(better, barg, rowstat_ref[:, 1:2])

            browid = jax.lax.broadcasted_iota(jnp.int32, (q, nblk), 0)
            cmax = jnp.max(s, axis=0, keepdims=True)
            carg = jnp.min(jnp.where(s == cmax, browid, q), axis=0,
                           keepdims=True)
            colarg_ref[0:1, pl.ds(j * nblk, nblk)] = carg

    @pl.when(i == ks * nb)
    def _finalize():
        sims = sims_ref[...]
        q, k = sims.shape
        ixs = rowstat_ref[:, 1:2].astype(jnp.int32)
        rowmax = rowstat_ref[:, 0:1]

        colid = jax.lax.broadcasted_iota(jnp.int32, (q, k), 1)
        onehot = colid == ixs
        pt1x = pt1t_ref[0:1, :]
        pt1y = pt1t_ref[1:2, :]
        pt1mx = jnp.max(jnp.where(onehot, pt1x, -1.0), axis=1, keepdims=True)
        pt1my = jnp.max(jnp.where(onehot, pt1y, -1.0), axis=1, keepdims=True)

        near = (jnp.abs(pt1x - pt1mx) < 64.0) & (jnp.abs(pt1y - pt1my) < 64.0)
        sim_rev = jnp.max(jnp.where(near, 0.0, sims), axis=1, keepdims=True)

        ix_cyc = jnp.min(jnp.where(onehot, colarg_ref[...], q), axis=1,
                         keepdims=True)

        qid = jax.lax.broadcasted_iota(jnp.int32, (q, q), 1)
        onehot2 = qid == ix_cyc
        pt0x = pt0t_ref[0:1, :]
        pt0y = pt0t_ref[1:2, :]
        pt0cx = jnp.max(jnp.where(onehot2, pt0x, -1.0), axis=1, keepdims=True)
        pt0cy = jnp.max(jnp.where(onehot2, pt0y, -1.0), axis=1, keepdims=True)
        dx = pt0cx - jax.lax.transpose(pt0x, (1, 0))
        dy = pt0cy - jax.lax.transpose(pt0y, (1, 0))

        out_ref[:, 0:1] = rowmax
        out_ref[:, 1:2] = rowmax / sim_rev
        out_ref[:, 2:3] = jnp.sqrt(dx * dx + dy * dy)
        out_ref[:, 3:4] = jax.lax.bitcast_convert_type(ixs, jnp.float32)


def kernel(ft0, ft1, pt0, pt1):
    nq, c, h, w = ft0.shape
    nk = ft1.shape[0]
    d = c * h * w
    scale = 1.0 / (h ** 2) / 2.0

    ft0t = jax.lax.transpose(ft0, (1, 2, 3, 0)).reshape(d, nq)
    ft1t = jax.lax.transpose(ft1, (1, 2, 3, 0)).reshape(d, nk)

    pt0t = pt0.astype(jnp.float32).T
    pt1t = pt1.astype(jnp.float32).T

    nblk = 1024 if nk % 1024 == 0 else nk
    nb = nk // nblk
    ks = 4
    dh = d // ks
    last = nb - 1

    out = pl.pallas_call(
        functools.partial(_fused_body, scale=scale, nblk=nblk, nb=nb, dh=dh,
                          ks=ks),
        grid=(ks * nb + 1,),
        in_specs=[
            pl.BlockSpec((d, nq), lambda i: (0, 0)),
            pl.BlockSpec(
                (dh, nblk),
                lambda i: (jax.lax.rem(i, ks),
                           jnp.minimum(jax.lax.div(i, ks), last))),
            pl.BlockSpec((2, nq), lambda i: (0, 0)),
            pl.BlockSpec((2, nk), lambda i: (0, 0)),
        ],
        out_specs=pl.BlockSpec((nq, 4), lambda i: (0, 0)),
        out_shape=jax.ShapeDtypeStruct((nq, 4), jnp.float32),
        scratch_shapes=[
            pltpu.VMEM((nq, nk), jnp.float32),
            pltpu.VMEM((nq, 2), jnp.float32),
            pltpu.VMEM((1, nk), jnp.int32),
        ],
    )(ft0t, ft1t, pt0t, pt1t)

    return (out[:, 0], out[:, 1], out[:, 2],
            jax.lax.bitcast_convert_type(out[:, 3], jnp.int32))
